# baseline (speedup 1.0000x reference)
import functools

import jax
import jax.numpy as jnp
from jax import lax
from jax.experimental import pallas as pl
from jax.experimental.pallas import tpu as pltpu
from jax.experimental.pallas import tpu_sc as plsc

_H = 16
_TS = 2048
_NB = 32
_WIDTH = 4352
_BASE = 2304
_BW = 4608
_BOFF = 2312
_THR = (12, 16, 23, 32, 46, 64, 91)


def _sc_bucketize_body(b1d_hbm, buf_v, sem):
    w = lax.axis_index("c") * 16 + lax.axis_index("s")
    for t in range(9):
        def chunk(q, carry, t=t):
            m = lax.iota(jnp.int32, 16) + jnp.full((16,), q * 144 + t * 16 - _BOFF,
                                                   jnp.int32)
            n = jnp.abs(m)
            big = jnp.full((16,), 8, jnp.int32)
            one = jnp.full((16,), 1, jnp.int32)
            zero = jnp.full((16,), 0, jnp.int32)
            for thr in _THR:
                big = big + lax.select(n >= jnp.full((16,), thr, jnp.int32), one, zero)
            bucket = lax.select(n < jnp.full((16,), 8, jnp.int32), n, big)
            bucket = bucket + lax.select(m > zero, jnp.full((16,), 16, jnp.int32), zero)
            buf_v[pl.ds(t * 16, 16)] = bucket
            return carry
        chunk_t = functools.partial(chunk, t=t)
        _ = chunk_t(w, 0)
    cp = pltpu.async_copy(buf_v, b1d_hbm.at[pl.ds(w * 144, 144)], sem)
    cp.wait()


def _sc_bucket_table():
    mesh = plsc.VectorSubcoreMesh(core_axis_name="c", subcore_axis_name="s")
    return pl.kernel(
        _sc_bucketize_body,
        mesh=mesh,
        out_type=jax.ShapeDtypeStruct((_BW,), jnp.int32),
        scratch_types=[
            pltpu.VMEM((144,), jnp.int32),
            pltpu.SemaphoreType.DMA,
        ],
    )()


def _tc_body(bias_smem, b_ref, out_ref, w_ref):
    h = pl.program_id(0)
    bucket = jnp.concatenate(
        [b_ref[pl.ds(0, 1), pl.ds(8 - s, _WIDTH)] for s in range(8)], axis=0
    )
    acc = jnp.zeros((8, _WIDTH), jnp.float32)
    for b in range(_NB):
        acc = jnp.where(bucket == b, bias_smem[b, h], acc)
    w_ref[0:8, :] = acc
    cur = 8
    while cur < 128:
        w_ref[pl.ds(cur, cur), pl.ds(cur, _WIDTH - cur)] = (
            w_ref[pl.ds(0, cur), pl.ds(0, _WIDTH - cur)]
        )
        cur *= 2
    for g in range(16):
        out_ref[0, 0, pl.ds(128 * g, 128), :] = w_ref[:, pl.ds(_BASE - 128 * g, _TS)]


def kernel(T, bias):
    del T
    b2d = _sc_bucket_table().reshape(1, _BW)
    return pl.pallas_call(
        _tc_body,
        grid=(_H,),
        in_specs=[
            pl.BlockSpec(memory_space=pltpu.SMEM),
            pl.BlockSpec((1, _BW), lambda h: (0, 0)),
        ],
        out_specs=pl.BlockSpec((1, 1, _TS, _TS), lambda h: (0, h, 0, 0)),
        out_shape=jax.ShapeDtypeStruct((1, _H, _TS, _TS), jnp.float32),
        scratch_shapes=[pltpu.VMEM((128, _WIDTH), jnp.float32)],
    )(bias, b2d)


# speedup vs baseline: 1.1095x; 1.1095x over previous
import math

import jax
import jax.numpy as jnp
from jax.experimental import pallas as pl
from jax.experimental.pallas import tpu as pltpu

_H = 16
_TS = 2048
_NB = 32
_RB = 512
_WIDTH = 4352
_BASE = 2304
_LOG_SCALE = 8.0 / math.log(128.0 / 8.0)


def _bias_tc_kernel(bias_smem, out_ref, w_ref):
    h = pl.program_id(0)
    ib = pl.program_id(1)

    @pl.when(ib == 0)
    def _build_w():
        k = jax.lax.broadcasted_iota(jnp.int32, (8, _WIDTH), 1)
        s = jax.lax.broadcasted_iota(jnp.int32, (8, _WIDTH), 0)
        d = k - s - _BASE
        n = jnp.abs(d)
        big = 8 + (jnp.log(n.astype(jnp.float32) * 0.125 + 1e-6) * _LOG_SCALE).astype(jnp.int32)
        big = jnp.minimum(big, 15)
        bucket = jnp.where(n < 8, n, big) + jnp.where(d > 0, 16, 0)
        acc = jnp.zeros((8, _WIDTH), jnp.float32)
        for b in range(_NB):
            acc = jnp.where(bucket == b, bias_smem[b, h], acc)
        w_ref[0:8, :] = acc
        cur = 8
        while cur < 128:
            w_ref[pl.ds(cur, cur), pl.ds(cur, _WIDTH - cur)] = (
                w_ref[pl.ds(0, cur), pl.ds(0, _WIDTH - cur)]
            )
            cur *= 2

    for g in range(_RB // 128):
        start = pl.multiple_of(_BASE - _RB * ib - 128 * g, 128)
        out_ref[0, 0, pl.ds(128 * g, 128), :] = w_ref[:, pl.ds(start, _TS)]


def kernel(T, bias):
    del T
    return pl.pallas_call(
        _bias_tc_kernel,
        grid=(_H, _TS // _RB),
        in_specs=[pl.BlockSpec(memory_space=pltpu.SMEM)],
        out_specs=pl.BlockSpec((1, 1, _RB, _TS), lambda h, ib: (0, h, ib, 0)),
        out_shape=jax.ShapeDtypeStruct((1, _H, _TS, _TS), jnp.float32),
        scratch_shapes=[pltpu.VMEM((128, _WIDTH), jnp.float32)],
    )(bias)


# speedup vs baseline: 1.2247x; 1.1038x over previous
import math

import jax
import jax.numpy as jnp
from jax.experimental import pallas as pl
from jax.experimental.pallas import tpu as pltpu

_H = 16
_TS = 2048
_NB = 32
_WIDTH = 4352
_BASE = 2304
_LOG_SCALE = 8.0 / math.log(128.0 / 8.0)


def _bias_tc_kernel(bias_smem, out_ref, w_ref):
    h = pl.program_id(0)
    k = jax.lax.broadcasted_iota(jnp.int32, (8, _WIDTH), 1)
    s = jax.lax.broadcasted_iota(jnp.int32, (8, _WIDTH), 0)
    d = k - s - _BASE
    n = jnp.abs(d)
    big = 8 + (jnp.log(n.astype(jnp.float32) * 0.125 + 1e-6) * _LOG_SCALE).astype(jnp.int32)
    big = jnp.minimum(big, 15)
    bucket = jnp.where(n < 8, n, big) + jnp.where(d > 0, 16, 0)
    acc = jnp.zeros((8, _WIDTH), jnp.float32)
    for b in range(_NB):
        acc = jnp.where(bucket == b, bias_smem[b, h], acc)
    w_ref[0:8, :] = acc
    cur = 8
    while cur < 128:
        w_ref[pl.ds(cur, cur), pl.ds(cur, _WIDTH - cur)] = (
            w_ref[pl.ds(0, cur), pl.ds(0, _WIDTH - cur)]
        )
        cur *= 2
    for g in range(16):
        out_ref[0, 0, pl.ds(128 * g, 128), :] = w_ref[:, pl.ds(_BASE - 128 * g, _TS)]


def kernel(T, bias):
    del T
    return pl.pallas_call(
        _bias_tc_kernel,
        grid=(_H,),
        in_specs=[pl.BlockSpec(memory_space=pltpu.SMEM)],
        out_specs=pl.BlockSpec((1, 1, _TS, _TS), lambda h: (0, h, 0, 0)),
        out_shape=jax.ShapeDtypeStruct((1, _H, _TS, _TS), jnp.float32),
        scratch_shapes=[pltpu.VMEM((128, _WIDTH), jnp.float32)],
        compiler_params=pltpu.CompilerParams(dimension_semantics=("parallel",)),
    )(bias)


# speedup vs baseline: 1.2395x; 1.0120x over previous
import math

import jax
import jax.numpy as jnp
from jax.experimental import pallas as pl
from jax.experimental.pallas import tpu as pltpu

_H = 16
_TS = 2048
_NB = 32
_WIDTH = 4352
_BASE = 2304
_LOG_SCALE = 8.0 / math.log(128.0 / 8.0)


def _copy(w2_ref, out_ref, sems, par, h, g):
    return pltpu.make_async_copy(
        w2_ref.at[par, :, pl.ds(_BASE - 128 * g, _TS)],
        out_ref.at[0, h, pl.ds(128 * g, 128), :],
        sems.at[par],
    )


def _bias_tc_kernel(bias_smem, out_ref, w2_ref, sems):
    h = pl.program_id(0)
    par = jax.lax.rem(h, 2)

    @pl.when(h >= 2)
    def _drain_prev():
        for g in range(16):
            _copy(w2_ref, out_ref, sems, par, h - 2, g).wait()

    k = jax.lax.broadcasted_iota(jnp.int32, (8, _WIDTH), 1)
    s = jax.lax.broadcasted_iota(jnp.int32, (8, _WIDTH), 0)
    d = k - s - _BASE
    n = jnp.abs(d)
    big = 8 + (jnp.log(n.astype(jnp.float32) * 0.125 + 1e-6) * _LOG_SCALE).astype(jnp.int32)
    big = jnp.minimum(big, 15)
    bucket = jnp.where(n < 8, n, big) + jnp.where(d > 0, 16, 0)
    acc = jnp.zeros((8, _WIDTH), jnp.float32)
    for b in range(_NB):
        acc = jnp.where(bucket == b, bias_smem[b, h], acc)
    wv = w2_ref.at[par]
    wv[0:8, :] = acc
    cur = 8
    while cur < 128:
        wv[pl.ds(cur, cur), pl.ds(cur, _WIDTH - cur)] = (
            wv[pl.ds(0, cur), pl.ds(0, _WIDTH - cur)]
        )
        cur *= 2
    for g in range(16):
        _copy(w2_ref, out_ref, sems, par, h, g).start()

    @pl.when(h == _H - 1)
    def _drain_tail():
        for g in range(16):
            _copy(w2_ref, out_ref, sems, 1, h, g).wait()
            _copy(w2_ref, out_ref, sems, 0, h - 1, g).wait()


def kernel(T, bias):
    del T
    return pl.pallas_call(
        _bias_tc_kernel,
        grid=(_H,),
        in_specs=[pl.BlockSpec(memory_space=pltpu.SMEM)],
        out_specs=pl.BlockSpec(memory_space=pl.ANY),
        out_shape=jax.ShapeDtypeStruct((1, _H, _TS, _TS), jnp.float32),
        scratch_shapes=[
            pltpu.VMEM((2, 128, _WIDTH), jnp.float32),
            pltpu.SemaphoreType.DMA((2,)),
        ],
    )(bias)
